# baseline (speedup 1.0000x reference)
import jax
import jax.numpy as jnp
from jax import lax
from jax.experimental import pallas as pl
from jax.experimental.pallas import tpu as pltpu
from jax.experimental.pallas import tpu_sc as plsc

B = 8
N = 20000
C = 92
ROWS = B * N
NW = 32

SC_ROWS = 83200
RT = ROWS - SC_ROWS
RPW = SC_ROWS // NW
CH = 200
NCH = RPW // CH
NGF = CH // 16

BIG = 1 << 30

TBLK = 1600
NTBLK = RT // TBLK


def _vgather(v, idx):
    return lax.gather(
        v, idx[:, None],
        dimension_numbers=lax.GatherDimensionNumbers(
            offset_dims=(), collapsed_slice_dims=(0,), start_index_map=(0,)),
        slice_sizes=(1,),
        mode=lax.GatherScatterMode.PROMISE_IN_BOUNDS,
    )


def _sc_body(logits_hbm, boxes_hbm, ts_hbm, scores_hbm, labels_hbm, bout_hbm,
             lbuf, bbuf, tsbuf, sbuf, labf, bobuf):
    wid = lax.axis_index("s") * 2 + lax.axis_index("c")

    iota = lax.iota(jnp.int32, 16)
    lane15 = iota == 15
    lane_lt4 = iota < 4
    rot = [(iota + sh) % 16 for sh in (8, 4, 2, 1)]
    c15 = iota * 0 + 15
    idx_c = [iota, iota + 16, iota + 32, iota + 48, iota + 64, iota + 76]

    patA = iota - (iota % 4) + (iota % 2)
    patB = patA + 2
    sgn = jnp.where((iota % 4) < 2, -0.5, 0.5)

    pltpu.sync_copy(ts_hbm, tsbuf)
    tsv = tsbuf[...]

    def bf(v, op):
        for r in rot:
            v = op(v, _vgather(v, r))
        return v

    def rows16(gbase, svec, lvec):
        for j in range(16):
            r = gbase + j
            x0 = lbuf[r, pl.ds(0, 16)]
            x1 = lbuf[r, pl.ds(16, 16)]
            x2 = lbuf[r, pl.ds(32, 16)]
            x3 = lbuf[r, pl.ds(48, 16)]
            x4 = lbuf[r, pl.ds(64, 16)]
            x5 = lbuf[r, pl.ds(76, 16)]
            x5m = jnp.where(lane15, -jnp.inf, x5)

            val = x0
            idxv = idx_c[0]
            for t, xk in ((1, x1), (2, x2), (3, x3), (4, x4), (5, x5m)):
                upd = xk > val
                val = jnp.where(upd, xk, val)
                idxv = jnp.where(upd, idx_c[t], idxv)

            m91v = bf(val, jnp.maximum)
            mallv = jnp.maximum(m91v, _vgather(x5, c15))

            e0 = jnp.exp(x0 - mallv)
            e1 = jnp.exp(x1 - mallv)
            e2 = jnp.exp(x2 - mallv)
            e3 = jnp.exp(x3 - mallv)
            e4 = jnp.exp(x4 - mallv)
            e5 = jnp.exp(x5 - mallv)
            e5s = jnp.where(lane_lt4, 0.0, e5)
            sv = bf(e0 + e1 + e2 + e3 + e4 + e5s, jnp.add)

            scorev = jnp.exp(m91v - mallv) / sv
            labv = bf(jnp.where(val == m91v, idxv, BIG), jnp.minimum)

            lane_j = iota == j
            svec = jnp.where(lane_j, scorev, svec)
            lvec = jnp.where(lane_j, labv, lvec)
        return svec, lvec

    zf = jnp.zeros((16,), jnp.float32)
    zi = jnp.zeros((16,), jnp.int32)

    def chunk_body(k, carry):
        g0 = RT + wid * RPW + k * CH
        img = g0 // N
        n0 = pl.multiple_of(g0 % N, 8)
        pltpu.sync_copy(logits_hbm.at[img, pl.ds(n0, CH)], lbuf)
        def group_body(g, carry2):
            gbase = g * 16
            svec, lvec = rows16(gbase, zf, zi)
            sbuf[pl.ds(gbase, 16)] = svec
            labf[pl.ds(gbase, 16)] = lvec
            return carry2

        lax.fori_loop(0, NGF, group_body, 0)
        if CH % 16:
            svec, lvec = rows16(CH - 16, zf, zi)
            sbuf[pl.ds(CH - 16, 16)] = svec
            labf[pl.ds(CH - 16, 16)] = lvec

        rbase = pl.multiple_of(wid * RPW + k * CH, 8)
        pltpu.sync_copy(sbuf, scores_hbm.at[pl.ds(rbase, CH)])
        pltpu.sync_copy(labf, labels_hbm.at[pl.ds(rbase, CH)])
        return carry

    lax.fori_loop(0, NCH, chunk_body, 0)

    def box_chunk(k, carry):
        g0 = wid * (ROWS // NW) + k * CH
        img = g0 // N
        fb0 = pl.multiple_of(g0 * 4, 8)
        pltpu.sync_copy(boxes_hbm.at[pl.ds(fb0, CH * 4)], bbuf)
        ts_idx = jnp.where((iota % 2) == 0, 2 * img + 1, 2 * img)
        scalev = _vgather(tsv, ts_idx).astype(jnp.float32)

        def box_step(q, carry3):
            fb = q * 16
            v = bbuf[pl.ds(fb, 16)]
            a = _vgather(v, patA)
            wh = _vgather(v, patB)
            bobuf[pl.ds(fb, 16)] = (a + sgn * wh) * scalev
            return carry3

        lax.fori_loop(0, CH // 4, box_step, 0)
        pltpu.sync_copy(bobuf, bout_hbm.at[pl.ds(fb0, CH * 4)])
        return carry

    lax.fori_loop(0, ROWS // NW // CH, box_chunk, 0)


def _tc_body(logits_ref, scores_ref, labels_ref):
    x = logits_ref[...]
    m = jnp.max(x, axis=-1, keepdims=True)
    e = jnp.exp(x - m)
    s = jnp.sum(e, axis=-1, keepdims=True)
    e91 = e[:, : C - 1]
    scores_ref[...] = (jnp.max(e91, axis=-1, keepdims=True) / s).T[None]
    labels_ref[...] = jnp.argmax(e91, axis=-1, keepdims=True).T[None]


@jax.jit
def kernel(pred_logits, pred_boxes, target_sizes):
    boxes_flat = pred_boxes.reshape(ROWS * 4)
    ts_flat = target_sizes.reshape(2 * B)

    mesh = plsc.VectorSubcoreMesh(core_axis_name="c", subcore_axis_name="s")
    f = pl.kernel(
        _sc_body,
        mesh=mesh,
        compiler_params=pltpu.CompilerParams(use_tc_tiling_on_sc=True),
        out_type=[
            jax.ShapeDtypeStruct((SC_ROWS,), jnp.float32),
            jax.ShapeDtypeStruct((SC_ROWS,), jnp.int32),
            jax.ShapeDtypeStruct((ROWS * 4,), jnp.float32),
        ],
        scratch_types=[
            pltpu.VMEM((CH, C), jnp.float32),
            pltpu.VMEM((CH * 4,), jnp.float32),
            pltpu.VMEM((2 * B,), jnp.int32),
            pltpu.VMEM((CH,), jnp.float32),
            pltpu.VMEM((CH,), jnp.int32),
            pltpu.VMEM((CH * 4,), jnp.float32),
        ],
    )
    scores_sc, labels_sc, boxes_sc = f(pred_logits, boxes_flat, ts_flat)

    logits2 = pred_logits.reshape(ROWS, C)
    scores_tc, labels_tc = pl.pallas_call(
        _tc_body,
        grid=(NTBLK,),
        in_specs=[pl.BlockSpec((TBLK, C), lambda i: (i, 0))],
        out_specs=[
            pl.BlockSpec((1, 1, TBLK), lambda i: (i, 0, 0)),
            pl.BlockSpec((1, 1, TBLK), lambda i: (i, 0, 0)),
        ],
        out_shape=[
            jax.ShapeDtypeStruct((NTBLK, 1, TBLK), jnp.float32),
            jax.ShapeDtypeStruct((NTBLK, 1, TBLK), jnp.int32),
        ],
    )(logits2)

    scores = jnp.concatenate([scores_tc.reshape(RT), scores_sc])
    labels = jnp.concatenate([labels_tc.reshape(RT), labels_sc])
    return (scores.reshape(B, N), labels.reshape(B, N),
            boxes_sc.reshape(B, N, 4))


# speedup vs baseline: 1.4369x; 1.4369x over previous
import jax
import jax.numpy as jnp
from jax import lax
from jax.experimental import pallas as pl
from jax.experimental.pallas import tpu as pltpu
from jax.experimental.pallas import tpu_sc as plsc

B = 8
N = 20000
C = 92
ROWS = B * N
NW = 32
RPW = ROWS // NW
WPI = 4
CH = 200
NCH = RPW // CH
NGF = CH // 16

BIG = 1 << 30

BBLK = 2000


def _vgather(v, idx):
    return lax.gather(
        v, idx[:, None],
        dimension_numbers=lax.GatherDimensionNumbers(
            offset_dims=(), collapsed_slice_dims=(0,), start_index_map=(0,)),
        slice_sizes=(1,),
        mode=lax.GatherScatterMode.PROMISE_IN_BOUNDS,
    )


def _sc_body(logits_hbm, scores_hbm, labels_hbm, lbuf, sbuf, labf):
    wid = lax.axis_index("s") * 2 + lax.axis_index("c")
    img = wid // WPI
    base_n = (wid % WPI) * RPW

    iota = lax.iota(jnp.int32, 16)
    lane15 = iota == 15
    lane_lt4 = iota < 4
    rot = [(iota + sh) % 16 for sh in (8, 4, 2, 1)]
    c15 = iota * 0 + 15
    idx_c = [iota, iota + 16, iota + 32, iota + 48, iota + 64, iota + 76]

    def bf(v, op):
        for r in rot:
            v = op(v, _vgather(v, r))
        return v

    def rows16(gbase, svec, lvec):
        for j in range(16):
            r = gbase + j
            x0 = lbuf[r, pl.ds(0, 16)]
            x1 = lbuf[r, pl.ds(16, 16)]
            x2 = lbuf[r, pl.ds(32, 16)]
            x3 = lbuf[r, pl.ds(48, 16)]
            x4 = lbuf[r, pl.ds(64, 16)]
            x5 = lbuf[r, pl.ds(76, 16)]
            x5m = jnp.where(lane15, -jnp.inf, x5)

            val = x0
            idxv = idx_c[0]
            for t, xk in ((1, x1), (2, x2), (3, x3), (4, x4), (5, x5m)):
                upd = xk > val
                val = jnp.where(upd, xk, val)
                idxv = jnp.where(upd, idx_c[t], idxv)

            m91v = bf(val, jnp.maximum)
            mallv = jnp.maximum(m91v, _vgather(x5, c15))

            e0 = jnp.exp(x0 - mallv)
            e1 = jnp.exp(x1 - mallv)
            e2 = jnp.exp(x2 - mallv)
            e3 = jnp.exp(x3 - mallv)
            e4 = jnp.exp(x4 - mallv)
            e5 = jnp.exp(x5 - mallv)
            e5s = jnp.where(lane_lt4, 0.0, e5)
            sv = bf(e0 + e1 + e2 + e3 + e4 + e5s, jnp.add)

            scorev = jnp.exp(m91v - mallv) / sv
            labv = bf(jnp.where(val == m91v, idxv, BIG), jnp.minimum)

            lane_j = iota == j
            svec = jnp.where(lane_j, scorev, svec)
            lvec = jnp.where(lane_j, labv, lvec)
        return svec, lvec

    zf = jnp.zeros((16,), jnp.float32)
    zi = jnp.zeros((16,), jnp.int32)

    def chunk_body(k, carry):
        n0 = pl.multiple_of(base_n + k * CH, 8)
        pltpu.sync_copy(logits_hbm.at[img, pl.ds(n0, CH)], lbuf)

        def group_body(g, carry2):
            gbase = g * 16
            svec, lvec = rows16(gbase, zf, zi)
            sbuf[pl.ds(gbase, 16)] = svec
            labf[pl.ds(gbase, 16)] = lvec
            return carry2

        lax.fori_loop(0, NGF, group_body, 0)
        if CH % 16:
            svec, lvec = rows16(CH - 16, zf, zi)
            sbuf[pl.ds(CH - 16, 16)] = svec
            labf[pl.ds(CH - 16, 16)] = lvec

        rbase = pl.multiple_of(wid * RPW + k * CH, 8)
        pltpu.sync_copy(sbuf, scores_hbm.at[pl.ds(rbase, CH)])
        pltpu.sync_copy(labf, labels_hbm.at[pl.ds(rbase, CH)])
        return carry

    lax.fori_loop(0, NCH, chunk_body, 0)


def _tc_boxes_body(ts_ref, boxes_ref, boxes_out_ref):
    i = pl.program_id(0)
    b = i // (N // BBLK)
    th = ts_ref[b, 0].astype(jnp.float32)
    tw = ts_ref[b, 1].astype(jnp.float32)
    bx = boxes_ref[...]
    cxcy = bx[:, 0:2]
    wh = bx[:, 2:4]
    lo = cxcy - 0.5 * wh
    hi = cxcy + 0.5 * wh
    sv = jnp.stack([tw, th, tw, th])
    boxes_out_ref[...] = jnp.concatenate([lo, hi], axis=1) * sv[None, :]


@jax.jit
def kernel(pred_logits, pred_boxes, target_sizes):
    mesh = plsc.VectorSubcoreMesh(core_axis_name="c", subcore_axis_name="s")
    f = pl.kernel(
        _sc_body,
        mesh=mesh,
        compiler_params=pltpu.CompilerParams(use_tc_tiling_on_sc=True),
        out_type=[
            jax.ShapeDtypeStruct((ROWS,), jnp.float32),
            jax.ShapeDtypeStruct((ROWS,), jnp.int32),
        ],
        scratch_types=[
            pltpu.VMEM((CH, C), jnp.float32),
            pltpu.VMEM((CH,), jnp.float32),
            pltpu.VMEM((CH,), jnp.int32),
        ],
    )
    scores, labels = f(pred_logits)

    boxes2 = pred_boxes.reshape(ROWS, 4)
    boxes = pl.pallas_call(
        _tc_boxes_body,
        grid=(ROWS // BBLK,),
        in_specs=[
            pl.BlockSpec(memory_space=pltpu.SMEM),
            pl.BlockSpec((BBLK, 4), lambda i: (i, 0)),
        ],
        out_specs=pl.BlockSpec((BBLK, 4), lambda i: (i, 0)),
        out_shape=jax.ShapeDtypeStruct((ROWS, 4), jnp.float32),
    )(target_sizes, boxes2)

    return (scores.reshape(B, N), labels.reshape(B, N),
            boxes.reshape(B, N, 4))


# speedup vs baseline: 1.5946x; 1.1098x over previous
import jax
import jax.numpy as jnp
from jax import lax
from jax.experimental import pallas as pl
from jax.experimental.pallas import tpu as pltpu
from jax.experimental.pallas import tpu_sc as plsc

B = 8
N = 20000
C = 92
ROWS = B * N
NW = 32
RPW = ROWS // NW
WPI = 4
CH = 200
NCH = RPW // CH
NGF = CH // 16

BIG = 1 << 30

BBLK = 2000


def _vgather(v, idx):
    return lax.gather(
        v, idx[:, None],
        dimension_numbers=lax.GatherDimensionNumbers(
            offset_dims=(), collapsed_slice_dims=(0,), start_index_map=(0,)),
        slice_sizes=(1,),
        mode=lax.GatherScatterMode.PROMISE_IN_BOUNDS,
    )


def _sc_body(logits_hbm, scores_hbm, labels_hbm, lbuf, sbuf, labf):
    wid = lax.axis_index("s") * 2 + lax.axis_index("c")
    img = wid // WPI
    base_n = (wid % WPI) * RPW

    iota = lax.iota(jnp.int32, 16)
    lane15 = iota == 15
    lane_lt4 = iota < 4
    rot = [(iota + sh) % 16 for sh in (8, 4, 2, 1)]
    c15 = iota * 0 + 15
    fiota = iota.astype(jnp.float32)
    idx_c = [fiota, fiota + 16.0, fiota + 32.0, fiota + 48.0, fiota + 64.0,
             fiota + 76.0]

    def bf(v, op):
        for r in rot:
            v = op(v, _vgather(v, r))
        return v

    def rows16(gbase, svec, lvec):
        for j in range(16):
            r = gbase + j
            x0 = lbuf[r, pl.ds(0, 16)]
            x1 = lbuf[r, pl.ds(16, 16)]
            x2 = lbuf[r, pl.ds(32, 16)]
            x3 = lbuf[r, pl.ds(48, 16)]
            x4 = lbuf[r, pl.ds(64, 16)]
            x5 = lbuf[r, pl.ds(76, 16)]
            x5m = jnp.where(lane15, -jnp.inf, x5)

            val = x0
            idxv = idx_c[0]
            for t, xk in ((1, x1), (2, x2), (3, x3), (4, x4), (5, x5m)):
                upd = xk > val
                val = jnp.where(upd, xk, val)
                idxv = jnp.where(upd, idx_c[t], idxv)

            m91v = bf(val, jnp.maximum)
            mallv = jnp.maximum(m91v, _vgather(x5, c15))

            e0 = jnp.exp(x0 - mallv)
            e1 = jnp.exp(x1 - mallv)
            e2 = jnp.exp(x2 - mallv)
            e3 = jnp.exp(x3 - mallv)
            e4 = jnp.exp(x4 - mallv)
            e5 = jnp.exp(x5 - mallv)
            e5s = jnp.where(lane_lt4, 0.0, e5)
            sv = bf(e0 + e1 + e2 + e3 + e4 + e5s, jnp.add)

            scorev = jnp.exp(m91v - mallv) / sv
            labv = bf(jnp.where(val == m91v, idxv, 1.0e9), jnp.minimum)

            lane_j = iota == j
            svec = jnp.where(lane_j, scorev, svec)
            lvec = jnp.where(lane_j, labv.astype(jnp.int32), lvec)
        return svec, lvec

    zf = jnp.zeros((16,), jnp.float32)
    zi = jnp.zeros((16,), jnp.int32)

    def chunk_body(k, carry):
        n0 = pl.multiple_of(base_n + k * CH, 8)
        pltpu.sync_copy(logits_hbm.at[img, pl.ds(n0, CH)], lbuf)

        def group_body(g, carry2):
            gbase = g * 16
            svec, lvec = rows16(gbase, zf, zi)
            sbuf[pl.ds(gbase, 16)] = svec
            labf[pl.ds(gbase, 16)] = lvec
            return carry2

        lax.fori_loop(0, NGF, group_body, 0)
        if CH % 16:
            svec, lvec = rows16(CH - 16, zf, zi)
            sbuf[pl.ds(CH - 16, 16)] = svec
            labf[pl.ds(CH - 16, 16)] = lvec

        rbase = pl.multiple_of(wid * RPW + k * CH, 8)
        pltpu.sync_copy(sbuf, scores_hbm.at[pl.ds(rbase, CH)])
        pltpu.sync_copy(labf, labels_hbm.at[pl.ds(rbase, CH)])
        return carry

    lax.fori_loop(0, NCH, chunk_body, 0)


def _tc_boxes_body(ts_ref, boxes_ref, boxes_out_ref):
    i = pl.program_id(0)
    b = i // (N // BBLK)
    th = ts_ref[b, 0].astype(jnp.float32)
    tw = ts_ref[b, 1].astype(jnp.float32)
    bx = boxes_ref[...]
    cxcy = bx[:, 0:2]
    wh = bx[:, 2:4]
    lo = cxcy - 0.5 * wh
    hi = cxcy + 0.5 * wh
    sv = jnp.stack([tw, th, tw, th])
    boxes_out_ref[...] = jnp.concatenate([lo, hi], axis=1) * sv[None, :]


@jax.jit
def kernel(pred_logits, pred_boxes, target_sizes):
    mesh = plsc.VectorSubcoreMesh(core_axis_name="c", subcore_axis_name="s")
    f = pl.kernel(
        _sc_body,
        mesh=mesh,
        compiler_params=pltpu.CompilerParams(use_tc_tiling_on_sc=True),
        out_type=[
            jax.ShapeDtypeStruct((ROWS,), jnp.float32),
            jax.ShapeDtypeStruct((ROWS,), jnp.int32),
        ],
        scratch_types=[
            pltpu.VMEM((CH, C), jnp.float32),
            pltpu.VMEM((CH,), jnp.float32),
            pltpu.VMEM((CH,), jnp.int32),
        ],
    )
    scores, labels = f(pred_logits)

    boxes2 = pred_boxes.reshape(ROWS, 4)
    boxes = pl.pallas_call(
        _tc_boxes_body,
        grid=(ROWS // BBLK,),
        in_specs=[
            pl.BlockSpec(memory_space=pltpu.SMEM),
            pl.BlockSpec((BBLK, 4), lambda i: (i, 0)),
        ],
        out_specs=pl.BlockSpec((BBLK, 4), lambda i: (i, 0)),
        out_shape=jax.ShapeDtypeStruct((ROWS, 4), jnp.float32),
    )(target_sizes, boxes2)

    return (scores.reshape(B, N), labels.reshape(B, N),
            boxes.reshape(B, N, 4))


# speedup vs baseline: 1.5993x; 1.0029x over previous
import jax
import jax.numpy as jnp
from jax import lax
from jax.experimental import pallas as pl
from jax.experimental.pallas import tpu as pltpu
from jax.experimental.pallas import tpu_sc as plsc

B = 8
N = 20000
C = 92
ROWS = B * N
NW = 32
RPW = ROWS // NW
WPI = 4
CH = 200
NCH = RPW // CH
NGF = CH // 16

BIG = 1 << 30

BBLK = 2000


def _vgather(v, idx):
    return lax.gather(
        v, idx[:, None],
        dimension_numbers=lax.GatherDimensionNumbers(
            offset_dims=(), collapsed_slice_dims=(0,), start_index_map=(0,)),
        slice_sizes=(1,),
        mode=lax.GatherScatterMode.PROMISE_IN_BOUNDS,
    )


def _sc_body(logits_hbm, scores_hbm, labels_hbm, lbuf, sbuf, labf):
    wid = lax.axis_index("s") * 2 + lax.axis_index("c")
    img = wid // WPI
    base_n = (wid % WPI) * RPW

    iota = lax.iota(jnp.int32, 16)
    lane15 = iota == 15
    lane_lt4 = iota < 4
    rot = [(iota + sh) % 16 for sh in (8, 4, 2, 1)]
    c15 = iota * 0 + 15
    fiota = iota.astype(jnp.float32)
    idx_c = [fiota, fiota + 16.0, fiota + 32.0, fiota + 48.0, fiota + 64.0,
             fiota + 76.0]

    def bf(v, op):
        for r in rot:
            v = op(v, _vgather(v, r))
        return v

    def rows16(gbase, svec, lvec):
        for j in range(16):
            r = gbase + j
            x0 = lbuf[r, pl.ds(0, 16)]
            x1 = lbuf[r, pl.ds(16, 16)]
            x2 = lbuf[r, pl.ds(32, 16)]
            x3 = lbuf[r, pl.ds(48, 16)]
            x4 = lbuf[r, pl.ds(64, 16)]
            x5 = lbuf[r, pl.ds(76, 16)]
            x5m = jnp.where(lane15, -jnp.inf, x5)

            val = x0
            idxv = idx_c[0]
            for t, xk in ((1, x1), (2, x2), (3, x3), (4, x4), (5, x5m)):
                upd = xk > val
                val = jnp.where(upd, xk, val)
                idxv = jnp.where(upd, idx_c[t], idxv)

            m91v = bf(val, jnp.maximum)
            mallv = jnp.maximum(m91v, _vgather(x5, c15))

            e0 = jnp.exp(x0 - mallv)
            e1 = jnp.exp(x1 - mallv)
            e2 = jnp.exp(x2 - mallv)
            e3 = jnp.exp(x3 - mallv)
            e4 = jnp.exp(x4 - mallv)
            e5 = jnp.exp(x5 - mallv)
            e5s = jnp.where(lane_lt4, 0.0, e5)
            sv = bf(e0 + e1 + e2 + e3 + e4 + e5s, jnp.add)

            scorev = jnp.exp(m91v - mallv) / sv
            labv = bf(jnp.where(val == m91v, idxv, 1.0e9), jnp.minimum)

            lane_j = iota == j
            svec = jnp.where(lane_j, scorev, svec)
            lvec = jnp.where(lane_j, labv.astype(jnp.int32), lvec)
        return svec, lvec

    zf = jnp.zeros((16,), jnp.float32)
    zi = jnp.zeros((16,), jnp.int32)

    def chunk_body(k, carry):
        n0 = pl.multiple_of(base_n + k * CH, 8)
        pltpu.sync_copy(logits_hbm.at[img, pl.ds(n0, CH)], lbuf)

        def group_body(g, carry2):
            gbase = g * 16
            svec, lvec = rows16(gbase, zf, zi)
            sbuf[pl.ds(gbase, 16)] = svec
            labf[pl.ds(gbase, 16)] = lvec
            return carry2

        lax.fori_loop(0, NGF, group_body, 0)
        if CH % 16:
            svec, lvec = rows16(CH - 16, zf, zi)
            sbuf[pl.ds(CH - 16, 16)] = svec
            labf[pl.ds(CH - 16, 16)] = lvec

        rbase = pl.multiple_of(wid * RPW + k * CH, 8)
        pltpu.sync_copy(sbuf, scores_hbm.at[pl.ds(rbase, CH)])
        pltpu.sync_copy(labf, labels_hbm.at[pl.ds(rbase, CH)])
        return carry

    lax.fori_loop(0, NCH, chunk_body, 0)


def _tc_boxes_body(ts_ref, boxes_ref, boxes_out_ref):
    i = pl.program_id(0)
    b = i // (N // BBLK)
    th = ts_ref[b, 0].astype(jnp.float32)
    tw = ts_ref[b, 1].astype(jnp.float32)
    bx = boxes_ref[...]
    cxcy = bx[:, 0:2]
    wh = bx[:, 2:4]
    lo = cxcy - 0.5 * wh
    hi = cxcy + 0.5 * wh
    sv = jnp.stack([tw, th, tw, th])
    boxes_out_ref[...] = jnp.concatenate([lo, hi], axis=1) * sv[None, :]


@jax.jit
def kernel(pred_logits, pred_boxes, target_sizes):
    mesh = plsc.VectorSubcoreMesh(core_axis_name="c", subcore_axis_name="s")
    f = pl.kernel(
        _sc_body,
        mesh=mesh,
        out_type=[
            jax.ShapeDtypeStruct((ROWS,), jnp.float32),
            jax.ShapeDtypeStruct((ROWS,), jnp.int32),
        ],
        scratch_types=[
            pltpu.VMEM((CH, C), jnp.float32),
            pltpu.VMEM((CH,), jnp.float32),
            pltpu.VMEM((CH,), jnp.int32),
        ],
    )
    scores, labels = f(pred_logits)

    boxes2 = pred_boxes.reshape(ROWS, 4)
    boxes = pl.pallas_call(
        _tc_boxes_body,
        grid=(ROWS // BBLK,),
        in_specs=[
            pl.BlockSpec(memory_space=pltpu.SMEM),
            pl.BlockSpec((BBLK, 4), lambda i: (i, 0)),
        ],
        out_specs=pl.BlockSpec((BBLK, 4), lambda i: (i, 0)),
        out_shape=jax.ShapeDtypeStruct((ROWS, 4), jnp.float32),
    )(target_sizes, boxes2)

    return (scores.reshape(B, N), labels.reshape(B, N),
            boxes.reshape(B, N, 4))
